# Initial kernel scaffold; baseline (speedup 1.0000x reference)
import jax
import jax.numpy as jnp
from jax.experimental import pallas as pl


def kernel(x, edge_index, edge_attr, batch, W1, b1, p1w, W2, b2, p2w, fc1W, fc1b, bn1g, bn1b, fc2W, fc2b, bn2g, bn2b, fc3W, fc3b, sW1, sb1, sg1, sbb1, sW2, sb2, sg2, sbb2, sW3, sb3):
    raise NotImplementedError("write your pallas kernel here")


# speedup vs baseline: 30.3898x; 30.3898x over previous
import functools

import jax
import jax.numpy as jnp
from jax import lax
from jax.experimental import pallas as pl
from jax.experimental.pallas import tpu as pltpu
from jax.experimental.pallas import tpu_sc as plsc

N = 10000
G = 100
NPG = 100
E = 320000
D_IN = 128
DIM = 64
K1 = 50
K2 = 25

NC = 2
NS = 16
NW = NC * NS
ROWLEN = 125
EROWS = E // ROWLEN
WROWS = EROWS // NW
KCH = 8
NBLK = WROWS // KCH
ACCR = 10240
RPT = ACCR // NS

_BIG = 10 ** 6
_NEG = float("-inf")

def _dot(a, b, dims):
    return lax.dot_general(a, b, (dims, ((), ())),
                           preferred_element_type=jnp.float32)


def _score(h, pw):
    hb = h.astype(jnp.bfloat16).astype(jnp.float32)
    pb = pw.astype(jnp.bfloat16).astype(jnp.float32)
    nrm = jnp.sqrt(jnp.sum(pw * pw, axis=1, keepdims=True))
    return jnp.tanh(jnp.sum(hb * pb, axis=1, keepdims=True) / nrm)


def _mm1_body(x_ref, w_ref, o_ref):
    o_ref[...] = _dot(x_ref[...], w_ref[...], (((1,), (1,))))


def _mid_body(a0, a1, b1, pw, h_ref, s_ref):
    h = jax.nn.relu(a0[...] + a1[...] + b1[...])
    h_ref[...] = h
    s_ref[...] = _score(h, pw[...])


def _pool_body(k, sg_ref, key_ref, v_ref, sel_ref, rank_ref):
    work = sg_ref[...]
    tiekey = key_ref[...]
    selm = jnp.zeros(work.shape, jnp.float32)
    rank = jnp.full(work.shape, _BIG, jnp.int32)
    big = jnp.int32(_BIG)
    for t in range(k):
        m = jnp.max(work, axis=1, keepdims=True)
        v_ref[:, t:t + 1] = jax.nn.sigmoid(m)
        eq = work == m
        kmin = jnp.min(jnp.where(eq, tiekey, big), axis=1, keepdims=True)
        onehot = eq & (tiekey == kmin)
        work = jnp.where(onehot, _NEG, work)
        selm = selm + onehot.astype(jnp.float32)
        rank = jnp.where(onehot, jnp.int32(t), rank)
    sel_ref[...] = selm
    rank_ref[...] = rank


def _stats1_body(h_ref, s_ref, sel_ref, w2_ref, x1_ref, hl2_ref):
    hp = h_ref[...] * (s_ref[...] * sel_ref[...])
    H3 = hp.reshape(G, NPG, DIM)
    sel3 = sel_ref[...].reshape(G, NPG, 1)
    x1_ref[:, 0:DIM] = jnp.max(jnp.where(sel3 > 0, H3, _NEG), axis=1)
    x1_ref[:, DIM:2 * DIM] = jnp.sum(H3, axis=1) * jnp.float32(1.0 / K1)
    hl2_ref[...] = _dot(hp, w2_ref[...], (((1,), (1,))))


def _fin_body(a0, a1, b2, pw, sel_ref, h_ref, s_ref, sm_ref):
    h = jax.nn.relu(a0[...] + a1[...] + b2[...])
    h_ref[...] = h
    s = _score(h, pw[...])
    s_ref[...] = s
    sm_ref[...] = jnp.where(sel_ref[...] > 0, s, _NEG)


def _heads_body(h_ref, s_ref, sel_ref, x1_ref,
                f1w, f1b, g1, c1, f2w, f2b, g2, c2, f3w, f3b,
                t1w, t1b, tg1, tc1, t2w, t2b, tg2, tc2, t3w, t3b,
                xy_ref, xs_ref):
    hq = h_ref[...] * (s_ref[...] * sel_ref[...])
    Q3 = hq.reshape(G, NPG, DIM)
    sel3 = sel_ref[...].reshape(G, NPG, 1)
    mx = jnp.max(jnp.where(sel3 > 0, Q3, _NEG), axis=1)
    mn = jnp.sum(Q3, axis=1) * jnp.float32(1.0 / K2)
    xg = jnp.concatenate([x1_ref[...], mx, mn], axis=1)
    ibn = jnp.float32(1.0) / jnp.sqrt(jnp.float32(1.0 + 1e-5))

    def fc(v, w, b):
        return _dot(v, w, (((1,), (1,)))) + b[...]

    def lsm(v):
        z = v - jnp.max(v, axis=1, keepdims=True)
        return z - jnp.log(jnp.sum(jnp.exp(z), axis=1, keepdims=True))

    y = jax.nn.relu(fc(xg, f1w[...], f1b)) * ibn * g1[...] + c1[...]
    y = jax.nn.relu(fc(y, f2w[...], f2b)) * ibn * g2[...] + c2[...]
    xy_ref[...] = lsm(fc(y, f3w[...], f3b))
    z = jax.nn.relu(fc(xg, t1w[...], t1b)) * ibn * tg1[...] + tc1[...]
    z = jax.nn.relu(fc(z, t2w[...], t2b)) * ibn * tg2[...] + tc2[...]
    xs_ref[...] = lsm(fc(z, t3w[...], t3b))


def _tc(body, out_shape):
    return pl.pallas_call(body, out_shape=out_shape)


def _sc_scatter_body(hlin, srcr, dstr, zer, out, sidx, didx, rows, obuf, acc,
                     sem):
    c = lax.axis_index("c")
    s = lax.axis_index("s")
    w = s * NC + c
    row0 = s * RPT
    pltpu.sync_copy(zer, obuf)
    pltpu.sync_copy(obuf, acc.at[pl.ds(row0, RPT)])
    plsc.subcore_barrier()

    def blk(b, carry):
        base = w * WROWS + b * KCH
        pltpu.sync_copy(srcr.at[pl.ds(base, KCH)], sidx)
        pltpu.sync_copy(dstr.at[pl.ds(base, KCH)], didx)
        for j in range(KCH):
            pltpu.async_copy(hlin.at[sidx.at[j]], rows, sem).wait()
            pltpu.sync_copy(rows, acc.at[didx.at[j]], add=True)
        return carry

    lax.fori_loop(0, NBLK, blk, 0)
    plsc.subcore_barrier()
    pltpu.sync_copy(acc.at[pl.ds(row0, RPT)], obuf)
    pltpu.sync_copy(obuf, out.at[c, pl.ds(row0, RPT)])


def _scatter_edges(hlin, src2d, dst2d, zer):
    mesh = plsc.VectorSubcoreMesh(core_axis_name="c", subcore_axis_name="s")
    f = pl.kernel(
        _sc_scatter_body,
        out_type=jax.ShapeDtypeStruct((NC, ACCR, DIM), jnp.float32),
        mesh=mesh,
        scratch_types=[
            pltpu.VMEM((KCH, ROWLEN), jnp.int32),
            pltpu.VMEM((KCH, ROWLEN), jnp.int32),
            pltpu.VMEM((ROWLEN, DIM), jnp.float32),
            pltpu.VMEM((RPT, DIM), jnp.float32),
            pltpu.VMEM_SHARED((ACCR, DIM), jnp.float32),
            pltpu.SemaphoreType.DMA,
        ],
        compiler_params=pltpu.CompilerParams(use_tc_tiling_on_sc=False),
    )
    return f(hlin, src2d, dst2d, zer)[:, :N]


def kernel(x, edge_index, edge_attr, batch, W1, b1, p1w, W2, b2, p2w,
           fc1W, fc1b, bn1g, bn1b, fc2W, fc2b, bn2g, bn2b, fc3W, fc3b,
           sW1, sb1, sg1, sbb1, sW2, sb2, sg2, sbb2, sW3, sb3):
    src2d = edge_index[0].reshape(EROWS, ROWLEN)
    dst2d = edge_index[1].reshape(EROWS, ROWLEN)
    zer = jnp.zeros((RPT, DIM), jnp.float32)
    r1 = lambda v: v.reshape(1, -1)

    hlin1 = _tc(_mm1_body, jax.ShapeDtypeStruct((N, DIM), jnp.float32))(x, W1)

    parts1 = _scatter_edges(hlin1, src2d, dst2d, zer)

    h1, s1 = _tc(_mid_body, (jax.ShapeDtypeStruct((N, DIM), jnp.float32),
                             jax.ShapeDtypeStruct((N, 1), jnp.float32)))(
        parts1[0], parts1[1], r1(b1), r1(p1w))

    iota = jnp.tile(jnp.arange(NPG, dtype=jnp.int32)[None, :], (G, 1))
    vals1, selm1, rank1 = _tc(
        functools.partial(_pool_body, K1),
        (jax.ShapeDtypeStruct((G, K1), jnp.float32),
         jax.ShapeDtypeStruct((G, NPG), jnp.float32),
         jax.ShapeDtypeStruct((G, NPG), jnp.int32)))(s1.reshape(G, NPG), iota)
    sel1 = selm1.reshape(N, 1)

    x1, hlin2 = _tc(_stats1_body,
                    (jax.ShapeDtypeStruct((G, 2 * DIM), jnp.float32),
                     jax.ShapeDtypeStruct((N, DIM), jnp.float32)))(
        h1, s1, sel1, W2)

    parts2 = _scatter_edges(hlin2, src2d, dst2d, zer)

    h2, s2, s2m = _tc(_fin_body, (jax.ShapeDtypeStruct((N, DIM), jnp.float32),
                                  jax.ShapeDtypeStruct((N, 1), jnp.float32),
                                  jax.ShapeDtypeStruct((N, 1), jnp.float32)))(
        parts2[0], parts2[1], r1(b2), r1(p2w), sel1)

    vals2, selm2, _ = _tc(
        functools.partial(_pool_body, K2),
        (jax.ShapeDtypeStruct((G, K2), jnp.float32),
         jax.ShapeDtypeStruct((G, NPG), jnp.float32),
         jax.ShapeDtypeStruct((G, NPG), jnp.int32)))(s2m.reshape(G, NPG), rank1)
    sel2 = selm2.reshape(N, 1)

    xy, xs = _tc(_heads_body, (jax.ShapeDtypeStruct((G, 2), jnp.float32),
                               jax.ShapeDtypeStruct((G, 20), jnp.float32)))(
        h2, s2, sel2, x1,
        fc1W, r1(fc1b), r1(bn1g), r1(bn1b),
        fc2W, r1(fc2b), r1(bn2g), r1(bn2b), fc3W, r1(fc3b),
        sW1, r1(sb1), r1(sg1), r1(sbb1),
        sW2, r1(sb2), r1(sg2), r1(sbb2), sW3, r1(sb3))

    return (xy, xs, vals1, vals2)


# speedup vs baseline: 43.3103x; 1.4252x over previous
import functools

import jax
import jax.numpy as jnp
from jax import lax
from jax.experimental import pallas as pl
from jax.experimental.pallas import tpu as pltpu
from jax.experimental.pallas import tpu_sc as plsc

N = 10000
G = 100
NPG = 100
E = 320000
D_IN = 128
DIM = 64
K1 = 50
K2 = 25

NC = 2
NS = 16
NW = NC * NS
ROWLEN = 125
EROWS = E // ROWLEN
WROWS = EROWS // NW
KCH = 8
NBLK = WROWS // KCH
ACCR = 10240
RPT = ACCR // NS

_BIG = 10 ** 6
_NEG = float("-inf")

def _dot(a, b, dims):
    return lax.dot_general(a, b, (dims, ((), ())),
                           preferred_element_type=jnp.float32)


def _score(h, pw):
    hb = h.astype(jnp.bfloat16).astype(jnp.float32)
    pb = pw.astype(jnp.bfloat16).astype(jnp.float32)
    nrm = jnp.sqrt(jnp.sum(pw * pw, axis=1, keepdims=True))
    return jnp.tanh(jnp.sum(hb * pb, axis=1, keepdims=True) / nrm)


def _mm1_body(x_ref, w_ref, o_ref):
    o_ref[...] = _dot(x_ref[...], w_ref[...], (((1,), (1,))))


def _mid_body(ap, b1, pw, h_ref, s_ref):
    h = jax.nn.relu(ap[0, :N, :] + ap[1, :N, :] + b1[...])
    h_ref[...] = h
    s_ref[...] = _score(h, pw[...])


def _pool_body(k, sg_ref, key_ref, v_ref, sel_ref, rank_ref):
    work = sg_ref[...]
    tiekey = key_ref[...]
    selm = jnp.zeros(work.shape, jnp.float32)
    rank = jnp.full(work.shape, _BIG, jnp.int32)
    big = jnp.int32(_BIG)
    for t in range(k):
        m = jnp.max(work, axis=1, keepdims=True)
        v_ref[:, t:t + 1] = jax.nn.sigmoid(m)
        eq = work == m
        kmin = jnp.min(jnp.where(eq, tiekey, big), axis=1, keepdims=True)
        onehot = eq & (tiekey == kmin)
        work = jnp.where(onehot, _NEG, work)
        selm = selm + onehot.astype(jnp.float32)
        rank = jnp.where(onehot, jnp.int32(t), rank)
    sel_ref[...] = selm
    rank_ref[...] = rank


def _stats1_body(h_ref, s_ref, sel_ref, w2_ref, x1_ref, hl2_ref):
    hp = h_ref[...] * (s_ref[...] * sel_ref[...])
    H3 = hp.reshape(G, NPG, DIM)
    sel3 = sel_ref[...].reshape(G, NPG, 1)
    x1_ref[:, 0:DIM] = jnp.max(jnp.where(sel3 > 0, H3, _NEG), axis=1)
    x1_ref[:, DIM:2 * DIM] = jnp.sum(H3, axis=1) * jnp.float32(1.0 / K1)
    hl2_ref[...] = _dot(hp, w2_ref[...], (((1,), (1,))))


def _fin_body(ap, b2, pw, sel_ref, h_ref, s_ref, sm_ref):
    h = jax.nn.relu(ap[0, :N, :] + ap[1, :N, :] + b2[...])
    h_ref[...] = h
    s = _score(h, pw[...])
    s_ref[...] = s
    sm_ref[...] = jnp.where(sel_ref[...] > 0, s, _NEG)


def _heads_body(h_ref, s_ref, sel_ref, x1_ref,
                f1w, f1b, g1, c1, f2w, f2b, g2, c2, f3w, f3b,
                t1w, t1b, tg1, tc1, t2w, t2b, tg2, tc2, t3w, t3b,
                xy_ref, xs_ref):
    hq = h_ref[...] * (s_ref[...] * sel_ref[...])
    Q3 = hq.reshape(G, NPG, DIM)
    sel3 = sel_ref[...].reshape(G, NPG, 1)
    mx = jnp.max(jnp.where(sel3 > 0, Q3, _NEG), axis=1)
    mn = jnp.sum(Q3, axis=1) * jnp.float32(1.0 / K2)
    xg = jnp.concatenate([x1_ref[...], mx, mn], axis=1)
    ibn = jnp.float32(1.0) / jnp.sqrt(jnp.float32(1.0 + 1e-5))

    def fc(v, w, b):
        return _dot(v, w, (((1,), (1,)))) + b[...]

    def lsm(v):
        z = v - jnp.max(v, axis=1, keepdims=True)
        return z - jnp.log(jnp.sum(jnp.exp(z), axis=1, keepdims=True))

    y = jax.nn.relu(fc(xg, f1w[...], f1b)) * ibn * g1[...] + c1[...]
    y = jax.nn.relu(fc(y, f2w[...], f2b)) * ibn * g2[...] + c2[...]
    xy_ref[...] = lsm(fc(y, f3w[...], f3b))
    z = jax.nn.relu(fc(xg, t1w[...], t1b)) * ibn * tg1[...] + tc1[...]
    z = jax.nn.relu(fc(z, t2w[...], t2b)) * ibn * tg2[...] + tc2[...]
    xs_ref[...] = lsm(fc(z, t3w[...], t3b))


def _tc(body, out_shape):
    return pl.pallas_call(body, out_shape=out_shape)


def _sc_scatter_body(hlin, srcr, dstr, zer, out, sidx, didx, rb0, rb1, obuf,
                     acc, sem0, sem1):
    c = lax.axis_index("c")
    s = lax.axis_index("s")
    w = s * NC + c
    row0 = s * RPT
    pltpu.sync_copy(zer, obuf)
    pltpu.sync_copy(obuf, acc.at[pl.ds(row0, RPT)])
    pltpu.sync_copy(srcr.at[pl.ds(w * WROWS, WROWS)], sidx)
    pltpu.sync_copy(dstr.at[pl.ds(w * WROWS, WROWS)], didx)
    plsc.subcore_barrier()

    pltpu.async_copy(hlin.at[sidx.at[0]], rb0, sem0)
    pltpu.async_copy(hlin.at[sidx.at[1]], rb1, sem1)

    def blk(t, carry):
        i = 2 * t
        pltpu.make_async_copy(hlin.at[sidx.at[i]], rb0, sem0).wait()
        pltpu.sync_copy(rb0, acc.at[didx.at[i]], add=True)

        @pl.when(i + 2 < WROWS)
        def _():
            pltpu.async_copy(hlin.at[sidx.at[i + 2]], rb0, sem0)

        pltpu.make_async_copy(hlin.at[sidx.at[i + 1]], rb1, sem1).wait()
        pltpu.sync_copy(rb1, acc.at[didx.at[i + 1]], add=True)

        @pl.when(i + 3 < WROWS)
        def _():
            pltpu.async_copy(hlin.at[sidx.at[i + 3]], rb1, sem1)

        return carry

    lax.fori_loop(0, WROWS // 2, blk, 0)
    plsc.subcore_barrier()
    pltpu.sync_copy(acc.at[pl.ds(row0, RPT)], obuf)
    pltpu.sync_copy(obuf, out.at[c, pl.ds(row0, RPT)])


def _scatter_edges(hlin, src2d, dst2d, zer):
    mesh = plsc.VectorSubcoreMesh(core_axis_name="c", subcore_axis_name="s")
    f = pl.kernel(
        _sc_scatter_body,
        out_type=jax.ShapeDtypeStruct((NC, ACCR, DIM), jnp.float32),
        mesh=mesh,
        scratch_types=[
            pltpu.VMEM((WROWS, ROWLEN), jnp.int32),
            pltpu.VMEM((WROWS, ROWLEN), jnp.int32),
            pltpu.VMEM((ROWLEN, DIM), jnp.float32),
            pltpu.VMEM((ROWLEN, DIM), jnp.float32),
            pltpu.VMEM((RPT, DIM), jnp.float32),
            pltpu.VMEM_SHARED((ACCR, DIM), jnp.float32),
            pltpu.SemaphoreType.DMA,
            pltpu.SemaphoreType.DMA,
        ],
        compiler_params=pltpu.CompilerParams(use_tc_tiling_on_sc=False),
    )
    return f(hlin, src2d, dst2d, zer)


def kernel(x, edge_index, edge_attr, batch, W1, b1, p1w, W2, b2, p2w,
           fc1W, fc1b, bn1g, bn1b, fc2W, fc2b, bn2g, bn2b, fc3W, fc3b,
           sW1, sb1, sg1, sbb1, sW2, sb2, sg2, sbb2, sW3, sb3):
    src2d = edge_index[0].reshape(EROWS, ROWLEN)
    dst2d = edge_index[1].reshape(EROWS, ROWLEN)
    zer = jnp.zeros((RPT, DIM), jnp.float32)
    r1 = lambda v: v.reshape(1, -1)

    hlin1 = _tc(_mm1_body, jax.ShapeDtypeStruct((N, DIM), jnp.float32))(x, W1)

    parts1 = _scatter_edges(hlin1, src2d, dst2d, zer)

    h1, s1 = _tc(_mid_body, (jax.ShapeDtypeStruct((N, DIM), jnp.float32),
                             jax.ShapeDtypeStruct((N, 1), jnp.float32)))(
        parts1, r1(b1), r1(p1w))

    iota = jnp.tile(jnp.arange(NPG, dtype=jnp.int32)[None, :], (G, 1))
    vals1, selm1, rank1 = _tc(
        functools.partial(_pool_body, K1),
        (jax.ShapeDtypeStruct((G, K1), jnp.float32),
         jax.ShapeDtypeStruct((G, NPG), jnp.float32),
         jax.ShapeDtypeStruct((G, NPG), jnp.int32)))(s1.reshape(G, NPG), iota)
    sel1 = selm1.reshape(N, 1)

    x1, hlin2 = _tc(_stats1_body,
                    (jax.ShapeDtypeStruct((G, 2 * DIM), jnp.float32),
                     jax.ShapeDtypeStruct((N, DIM), jnp.float32)))(
        h1, s1, sel1, W2)

    parts2 = _scatter_edges(hlin2, src2d, dst2d, zer)

    h2, s2, s2m = _tc(_fin_body, (jax.ShapeDtypeStruct((N, DIM), jnp.float32),
                                  jax.ShapeDtypeStruct((N, 1), jnp.float32),
                                  jax.ShapeDtypeStruct((N, 1), jnp.float32)))(
        parts2, r1(b2), r1(p2w), sel1)

    vals2, selm2, _ = _tc(
        functools.partial(_pool_body, K2),
        (jax.ShapeDtypeStruct((G, K2), jnp.float32),
         jax.ShapeDtypeStruct((G, NPG), jnp.float32),
         jax.ShapeDtypeStruct((G, NPG), jnp.int32)))(s2m.reshape(G, NPG), rank1)
    sel2 = selm2.reshape(N, 1)

    xy, xs = _tc(_heads_body, (jax.ShapeDtypeStruct((G, 2), jnp.float32),
                               jax.ShapeDtypeStruct((G, 20), jnp.float32)))(
        h2, s2, sel2, x1,
        fc1W, r1(fc1b), r1(bn1g), r1(bn1b),
        fc2W, r1(fc2b), r1(bn2g), r1(bn2b), fc3W, r1(fc3b),
        sW1, r1(sb1), r1(sg1), r1(sbb1),
        sW2, r1(sb2), r1(sg2), r1(sbb2), sW3, r1(sb3))

    return (xy, xs, vals1, vals2)


# speedup vs baseline: 53.0799x; 1.2256x over previous
import jax
import jax.numpy as jnp
from jax import lax
from jax.experimental import pallas as pl
from jax.experimental.pallas import tpu as pltpu
from jax.experimental.pallas import tpu_sc as plsc

N = 10000
G = 100
NPG = 100
E = 320000
D_IN = 128
DIM = 64
K1 = 50
K2 = 25

NC = 2
NS = 16
NW = NC * NS
ROWLEN = 125
EROWS = E // ROWLEN
WROWS = EROWS // NW
KCH = 8
NBLK = WROWS // KCH
ACCR = 10240
RPT = ACCR // NS
OBR = RPT // 2

_BIG = 10 ** 6
_NEG = float("-inf")

def _dot(a, b, dims):
    return lax.dot_general(a, b, (dims, ((), ())),
                           preferred_element_type=jnp.float32)


def _score(h, pw):
    hb = h.astype(jnp.bfloat16).astype(jnp.float32)
    pb = pw.astype(jnp.bfloat16).astype(jnp.float32)
    nrm = jnp.sqrt(jnp.sum(pw * pw, axis=1, keepdims=True))
    return jnp.sum(hb * pb, axis=1, keepdims=True), nrm


def _mm1_body(x_ref, w_ref, o_ref):
    o_ref[...] = _dot(x_ref[...], w_ref[...], (((1,), (1,))))


LW = 128


def _sort_desc(arrs, win):
    iota = lax.broadcasted_iota(jnp.int32, arrs[0].shape, 1)
    k = 2
    while k <= LW:
        j = k // 2
        while j >= 1:
            bitj = (iota & j) != 0
            P = [jnp.where(bitj, pltpu.roll(a, j, axis=1),
                           pltpu.roll(a, LW - j, axis=1)) for a in arrs]
            sw = win(arrs, P)
            take_w = bitj == ((iota & k) != 0)
            keep = take_w == sw
            arrs = [jnp.where(keep, a, p) for a, p in zip(arrs, P)]
            j //= 2
        k *= 2
    return arrs


def _tokey(s):
    b = lax.bitcast_convert_type(s, jnp.int32)
    return jnp.where(b < 0, b ^ jnp.int32(0x7FFFFFFF), b)


def _fromkey(k):
    b = jnp.where(k < 0, k ^ jnp.int32(0x7FFFFFFF), k)
    return lax.bitcast_convert_type(b, jnp.float32)


def _pad_min(x):
    return jnp.concatenate(
        [x, jnp.full((x.shape[0], LW - NPG), jnp.int32(-2 ** 31))], axis=1)


GB = 5
GPB = G // GB
RPB = GPB * NPG


def _stage1_body(ap, b1, pw, w2,
                 v_ref, selm_ref, sg_ref, x1_ref, hl2_ref):
    h = jax.nn.relu(ap[0] + ap[1] + b1[...])
    pre, nrm = _score(h, pw[...])
    sg_ref[0] = jnp.tanh(pre / nrm).reshape(GPB, NPG)
    sg = sg_ref[0]
    iota = lax.broadcasted_iota(jnp.int32, (GPB, LW), 1)
    kg = _tokey(sg)

    def win(a, b):
        return (a[0] > b[0]) | ((a[0] == b[0]) & (a[1] < b[1]))

    ks, si = _sort_desc([_pad_min(kg), iota], win)
    v_ref[0] = jax.nn.sigmoid(_fromkey(ks[:, :K1]))
    t_k = ks[:, K1 - 1:K1]
    t_i = si[:, K1 - 1:K1]
    idc = iota[:, :NPG]
    selm = ((kg > t_k) | ((kg == t_k) & (idc <= t_i))).astype(jnp.float32)
    selm_ref[0] = selm
    H3 = h.reshape(GPB, NPG, DIM)
    hp3 = H3 * (sg * selm).reshape(GPB, NPG, 1)
    sel3 = selm.reshape(GPB, NPG, 1)
    x1_ref[0, :, 0:DIM] = jnp.max(jnp.where(sel3 > 0, hp3, _NEG), axis=1)
    x1_ref[0, :, DIM:2 * DIM] = jnp.sum(hp3, axis=1) * jnp.float32(1.0 / K1)
    hl2_ref[...] = _dot(hp3.reshape(RPB, DIM), w2[...], (((1,), (1,))))


def _stage2_body(ap, b2, pw, selm1_ref, sg1_ref, x1_ref,
                 f1w, f1b, g1, c1, f2w, f2b, g2, c2, f3w, f3b,
                 t1w, t1b, tg1, tc1, t2w, t2b, tg2, tc2, t3w, t3b,
                 v_ref, xy_ref, xs_ref, scr_ref):
    h = jax.nn.relu(ap[0] + ap[1] + b2[...])
    pre, nrm = _score(h, pw[...])
    scr_ref[...] = jnp.tanh(pre / nrm).reshape(GPB, NPG)
    sg = scr_ref[...]
    sg1 = sg1_ref[0]
    sm = jnp.where(selm1_ref[0] > 0, sg, _NEG)
    iota = lax.broadcasted_iota(jnp.int32, (GPB, LW), 1)
    km = _tokey(sm)
    k1g = _tokey(sg1)

    def win(a, b):
        return ((a[0] > b[0])
                | ((a[0] == b[0])
                   & ((a[1] > b[1])
                      | ((a[1] == b[1]) & (a[2] < b[2])))))

    ks, k1s, si = _sort_desc([_pad_min(km), _pad_min(k1g), iota], win)
    v_ref[0] = jax.nn.sigmoid(_fromkey(ks[:, :K2]))
    t_k = ks[:, K2 - 1:K2]
    t_1 = k1s[:, K2 - 1:K2]
    t_i = si[:, K2 - 1:K2]
    idc = iota[:, :NPG]
    selm2 = ((km > t_k)
             | ((km == t_k)
                & ((k1g > t_1)
                   | ((k1g == t_1) & (idc <= t_i))))).astype(jnp.float32)
    Q3 = h.reshape(GPB, NPG, DIM) * (sg * selm2).reshape(GPB, NPG, 1)
    sel3 = selm2.reshape(GPB, NPG, 1)
    mx = jnp.max(jnp.where(sel3 > 0, Q3, _NEG), axis=1)
    mn = jnp.sum(Q3, axis=1) * jnp.float32(1.0 / K2)
    xg = jnp.concatenate([x1_ref[0], mx, mn], axis=1)
    ibn = jnp.float32(1.0) / jnp.sqrt(jnp.float32(1.0 + 1e-5))

    def fc(v, w, b):
        return _dot(v, w, (((1,), (1,)))) + b[...]

    def lsm(v):
        z = v - jnp.max(v, axis=1, keepdims=True)
        return z - jnp.log(jnp.sum(jnp.exp(z), axis=1, keepdims=True))

    y = jax.nn.relu(fc(xg, f1w[...], f1b)) * ibn * g1[...] + c1[...]
    y = jax.nn.relu(fc(y, f2w[...], f2b)) * ibn * g2[...] + c2[...]
    xy_ref[0] = lsm(fc(y, f3w[...], f3b))
    z = jax.nn.relu(fc(xg, t1w[...], t1b)) * ibn * tg1[...] + tc1[...]
    z = jax.nn.relu(fc(z, t2w[...], t2b)) * ibn * tg2[...] + tc2[...]
    xs_ref[0] = lsm(fc(z, t3w[...], t3b))


def _tc(body, out_shape):
    return pl.pallas_call(body, out_shape=out_shape)


NBUF = 4


def _sc_scatter_body(hlin, srcr, dstr, zer, out, sidx, didx, rb0, rb1, rb2,
                     rb3, obuf, acc, gs0, gs1, gs2, gs3, ss0, ss1, ss2, ss3):
    c = lax.axis_index("c")
    s = lax.axis_index("s")
    w = s * NC + c
    row0 = s * RPT
    rbs = (rb0, rb1, rb2, rb3)
    gss = (gs0, gs1, gs2, gs3)
    sss = (ss0, ss1, ss2, ss3)
    pltpu.sync_copy(zer, obuf)
    pltpu.sync_copy(obuf, acc.at[pl.ds(row0, OBR)])
    pltpu.sync_copy(obuf, acc.at[pl.ds(row0 + OBR, OBR)])
    pltpu.sync_copy(srcr.at[pl.ds(w * WROWS, WROWS)], sidx)
    pltpu.sync_copy(dstr.at[pl.ds(w * WROWS, WROWS)], didx)
    plsc.subcore_barrier()

    for k in range(NBUF):
        pltpu.async_copy(hlin.at[sidx.at[k]], rbs[k], gss[k])

    def blk(t, carry):
        i = NBUF * t
        for k in range(NBUF):
            pltpu.make_async_copy(hlin.at[sidx.at[i + k]], rbs[k],
                                  gss[k]).wait()
            pltpu.async_copy(rbs[k], acc.at[didx.at[i + k]], sss[k], add=True)
        for k in range(NBUF):
            pltpu.make_async_copy(rbs[k], acc.at[didx.at[i + k]],
                                  sss[k]).wait()

            @pl.when(i + NBUF + k < WROWS)
            def _():
                pltpu.async_copy(hlin.at[sidx.at[i + NBUF + k]], rbs[k],
                                 gss[k])

        return carry

    lax.fori_loop(0, WROWS // NBUF, blk, 0)
    plsc.subcore_barrier()
    pltpu.sync_copy(acc.at[pl.ds(row0, OBR)], obuf)
    pltpu.sync_copy(obuf, out.at[c, pl.ds(row0, OBR)])
    pltpu.sync_copy(acc.at[pl.ds(row0 + OBR, OBR)], obuf)
    pltpu.sync_copy(obuf, out.at[c, pl.ds(row0 + OBR, OBR)])


def _scatter_edges(hlin, src2d, dst2d, zer):
    mesh = plsc.VectorSubcoreMesh(core_axis_name="c", subcore_axis_name="s")
    f = pl.kernel(
        _sc_scatter_body,
        out_type=jax.ShapeDtypeStruct((NC, ACCR, DIM), jnp.float32),
        mesh=mesh,
        scratch_types=[
            pltpu.VMEM((WROWS, ROWLEN), jnp.int32),
            pltpu.VMEM((WROWS, ROWLEN), jnp.int32),
            pltpu.VMEM((ROWLEN, DIM), jnp.float32),
            pltpu.VMEM((ROWLEN, DIM), jnp.float32),
            pltpu.VMEM((ROWLEN, DIM), jnp.float32),
            pltpu.VMEM((ROWLEN, DIM), jnp.float32),
            pltpu.VMEM((OBR, DIM), jnp.float32),
            pltpu.VMEM_SHARED((ACCR, DIM), jnp.float32),
            pltpu.SemaphoreType.DMA,
            pltpu.SemaphoreType.DMA,
            pltpu.SemaphoreType.DMA,
            pltpu.SemaphoreType.DMA,
            pltpu.SemaphoreType.DMA,
            pltpu.SemaphoreType.DMA,
            pltpu.SemaphoreType.DMA,
            pltpu.SemaphoreType.DMA,
        ],
        compiler_params=pltpu.CompilerParams(use_tc_tiling_on_sc=False),
    )
    return f(hlin, src2d, dst2d, zer)


def kernel(x, edge_index, edge_attr, batch, W1, b1, p1w, W2, b2, p2w,
           fc1W, fc1b, bn1g, bn1b, fc2W, fc2b, bn2g, bn2b, fc3W, fc3b,
           sW1, sb1, sg1, sbb1, sW2, sb2, sg2, sbb2, sW3, sb3):
    src2d = edge_index[0].reshape(EROWS, ROWLEN)
    dst2d = edge_index[1].reshape(EROWS, ROWLEN)
    zer = jnp.zeros((OBR, DIM), jnp.float32)
    r1 = lambda v: v.reshape(1, -1)

    hlin1 = _tc(_mm1_body, jax.ShapeDtypeStruct((N, DIM), jnp.float32))(x, W1)

    parts1 = _scatter_edges(hlin1, src2d, dst2d, zer)

    def _full(shape):
        return pl.BlockSpec(shape, lambda i: tuple(0 for _ in shape))

    def _gblk(minor):
        return pl.BlockSpec((1, GPB, minor), lambda i: (i, 0, 0))

    ap_spec = pl.BlockSpec((2, RPB, DIM), lambda i: (0, i, 0))

    vals1, selm1, sc1, x1, hlin2 = pl.pallas_call(
        _stage1_body,
        grid=(GB,),
        in_specs=[ap_spec, _full((1, DIM)), _full((1, DIM)),
                  _full((DIM, DIM))],
        out_specs=[_gblk(K1), _gblk(NPG), _gblk(NPG), _gblk(2 * DIM),
                   pl.BlockSpec((RPB, DIM), lambda i: (i, 0))],
        out_shape=(
            jax.ShapeDtypeStruct((GB, GPB, K1), jnp.float32),
            jax.ShapeDtypeStruct((GB, GPB, NPG), jnp.float32),
            jax.ShapeDtypeStruct((GB, GPB, NPG), jnp.float32),
            jax.ShapeDtypeStruct((GB, GPB, 2 * DIM), jnp.float32),
            jax.ShapeDtypeStruct((N, DIM), jnp.float32)))(
        parts1, r1(b1), r1(p1w), W2)

    parts2 = _scatter_edges(hlin2, src2d, dst2d, zer)

    vals2, xy, xs = pl.pallas_call(
        _stage2_body,
        grid=(GB,),
        scratch_shapes=[pltpu.VMEM((GPB, NPG), jnp.float32)],
        in_specs=[ap_spec, _full((1, DIM)), _full((1, DIM)),
                  _gblk(NPG), _gblk(NPG), _gblk(2 * DIM),
                  _full(fc1W.shape), _full((1, fc1b.shape[0])),
                  _full((1, bn1g.shape[0])), _full((1, bn1b.shape[0])),
                  _full(fc2W.shape), _full((1, fc2b.shape[0])),
                  _full((1, bn2g.shape[0])), _full((1, bn2b.shape[0])),
                  _full(fc3W.shape), _full((1, fc3b.shape[0])),
                  _full(sW1.shape), _full((1, sb1.shape[0])),
                  _full((1, sg1.shape[0])), _full((1, sbb1.shape[0])),
                  _full(sW2.shape), _full((1, sb2.shape[0])),
                  _full((1, sg2.shape[0])), _full((1, sbb2.shape[0])),
                  _full(sW3.shape), _full((1, sb3.shape[0]))],
        out_specs=[_gblk(K2), _gblk(2), _gblk(20)],
        out_shape=(
            jax.ShapeDtypeStruct((GB, GPB, K2), jnp.float32),
            jax.ShapeDtypeStruct((GB, GPB, 2), jnp.float32),
            jax.ShapeDtypeStruct((GB, GPB, 20), jnp.float32)))(
        parts2, r1(b2), r1(p2w), selm1, sc1, x1,
        fc1W, r1(fc1b), r1(bn1g), r1(bn1b),
        fc2W, r1(fc2b), r1(bn2g), r1(bn2b), fc3W, r1(fc3b),
        sW1, r1(sb1), r1(sg1), r1(sbb1),
        sW2, r1(sb2), r1(sg2), r1(sbb2), sW3, r1(sb3))

    return (xy.reshape(G, 2), xs.reshape(G, 20),
            vals1.reshape(G, K1), vals2.reshape(G, K2))


# speedup vs baseline: 56.4292x; 1.0631x over previous
import jax
import jax.numpy as jnp
from jax import lax
from jax.experimental import pallas as pl
from jax.experimental.pallas import tpu as pltpu
from jax.experimental.pallas import tpu_sc as plsc

N = 10000
G = 100
NPG = 100
E = 320000
D_IN = 128
DIM = 64
K1 = 50
K2 = 25

NC = 2
NS = 16
NW = NC * NS
ROWLEN = 125
EROWS = E // ROWLEN
WROWS = EROWS // NW
KCH = 8
NBLK = WROWS // KCH
ACCR = 10240
RPT = ACCR // NS
OBR = RPT // 2

_BIG = 10 ** 6
_NEG = float("-inf")

def _dot(a, b, dims):
    return lax.dot_general(a, b, (dims, ((), ())),
                           preferred_element_type=jnp.float32)


def _score(h, pw):
    hb = h.astype(jnp.bfloat16).astype(jnp.float32)
    pb = pw.astype(jnp.bfloat16).astype(jnp.float32)
    nrm = jnp.sqrt(jnp.sum(pw * pw, axis=1, keepdims=True))
    return jnp.sum(hb * pb, axis=1, keepdims=True), nrm


def _mm1_body(x_ref, w_ref, o_ref):
    o_ref[...] = _dot(x_ref[...], w_ref[...], (((1,), (1,))))


LW = 128


def _sort_desc(arrs, win):
    iota = lax.broadcasted_iota(jnp.int32, arrs[0].shape, 1)
    k = 2
    while k <= LW:
        j = k // 2
        while j >= 1:
            bitj = (iota & j) != 0
            P = [jnp.where(bitj, pltpu.roll(a, j, axis=1),
                           pltpu.roll(a, LW - j, axis=1)) for a in arrs]
            sw = win(arrs, P)
            take_w = bitj == ((iota & k) != 0)
            keep = take_w == sw
            arrs = [jnp.where(keep, a, p) for a, p in zip(arrs, P)]
            j //= 2
        k *= 2
    return arrs


def _tokey(s):
    b = lax.bitcast_convert_type(s, jnp.int32)
    return jnp.where(b < 0, b ^ jnp.int32(0x7FFFFFFF), b)


def _fromkey(k):
    b = jnp.where(k < 0, k ^ jnp.int32(0x7FFFFFFF), k)
    return lax.bitcast_convert_type(b, jnp.float32)


def _pad_min(x):
    return jnp.concatenate(
        [x, jnp.full((x.shape[0], LW - NPG), jnp.int32(-2 ** 31))], axis=1)


GB = 1
GPB = G // GB
RPB = GPB * NPG


def _stage1_body(ap, b1, pw, w2,
                 v_ref, selm_ref, sg_ref, x1_ref, hl2_ref):
    h = jax.nn.relu(ap[0] + ap[1] + b1[...])
    pre, nrm = _score(h, pw[...])
    sg_ref[0] = jnp.tanh(pre / nrm).reshape(GPB, NPG)
    sg = sg_ref[0]
    iota = lax.broadcasted_iota(jnp.int32, (GPB, LW), 1)
    kg = _tokey(sg)

    def win(a, b):
        return (a[0] > b[0]) | ((a[0] == b[0]) & (a[1] < b[1]))

    ks, si = _sort_desc([_pad_min(kg), iota], win)
    v_ref[0] = jax.nn.sigmoid(_fromkey(ks[:, :K1]))
    t_k = ks[:, K1 - 1:K1]
    t_i = si[:, K1 - 1:K1]
    idc = iota[:, :NPG]
    selm = ((kg > t_k) | ((kg == t_k) & (idc <= t_i))).astype(jnp.float32)
    selm_ref[0] = selm
    H3 = h.reshape(GPB, NPG, DIM)
    hp3 = H3 * (sg * selm).reshape(GPB, NPG, 1)
    sel3 = selm.reshape(GPB, NPG, 1)
    x1_ref[0, :, 0:DIM] = jnp.max(jnp.where(sel3 > 0, hp3, _NEG), axis=1)
    x1_ref[0, :, DIM:2 * DIM] = jnp.sum(hp3, axis=1) * jnp.float32(1.0 / K1)
    hl2_ref[...] = _dot(hp3.reshape(RPB, DIM), w2[...], (((1,), (1,))))


def _stage2_body(ap, b2, pw, selm1_ref, sg1_ref, x1_ref,
                 f1w, f1b, g1, c1, f2w, f2b, g2, c2, f3w, f3b,
                 t1w, t1b, tg1, tc1, t2w, t2b, tg2, tc2, t3w, t3b,
                 v_ref, xy_ref, xs_ref, scr_ref):
    h = jax.nn.relu(ap[0] + ap[1] + b2[...])
    pre, nrm = _score(h, pw[...])
    scr_ref[...] = jnp.tanh(pre / nrm).reshape(GPB, NPG)
    sg = scr_ref[...]
    sg1 = sg1_ref[0]
    sm = jnp.where(selm1_ref[0] > 0, sg, _NEG)
    iota = lax.broadcasted_iota(jnp.int32, (GPB, LW), 1)
    km = _tokey(sm)
    k1g = _tokey(sg1)

    def win(a, b):
        return ((a[0] > b[0])
                | ((a[0] == b[0])
                   & ((a[1] > b[1])
                      | ((a[1] == b[1]) & (a[2] < b[2])))))

    ks, k1s, si = _sort_desc([_pad_min(km), _pad_min(k1g), iota], win)
    v_ref[0] = jax.nn.sigmoid(_fromkey(ks[:, :K2]))
    t_k = ks[:, K2 - 1:K2]
    t_1 = k1s[:, K2 - 1:K2]
    t_i = si[:, K2 - 1:K2]
    idc = iota[:, :NPG]
    selm2 = ((km > t_k)
             | ((km == t_k)
                & ((k1g > t_1)
                   | ((k1g == t_1) & (idc <= t_i))))).astype(jnp.float32)
    Q3 = h.reshape(GPB, NPG, DIM) * (sg * selm2).reshape(GPB, NPG, 1)
    sel3 = selm2.reshape(GPB, NPG, 1)
    mx = jnp.max(jnp.where(sel3 > 0, Q3, _NEG), axis=1)
    mn = jnp.sum(Q3, axis=1) * jnp.float32(1.0 / K2)
    xg = jnp.concatenate([x1_ref[0], mx, mn], axis=1)
    ibn = jnp.float32(1.0) / jnp.sqrt(jnp.float32(1.0 + 1e-5))

    def fc(v, w, b):
        return _dot(v, w, (((1,), (1,)))) + b[...]

    def lsm(v):
        z = v - jnp.max(v, axis=1, keepdims=True)
        return z - jnp.log(jnp.sum(jnp.exp(z), axis=1, keepdims=True))

    y = jax.nn.relu(fc(xg, f1w[...], f1b)) * ibn * g1[...] + c1[...]
    y = jax.nn.relu(fc(y, f2w[...], f2b)) * ibn * g2[...] + c2[...]
    xy_ref[0] = lsm(fc(y, f3w[...], f3b))
    z = jax.nn.relu(fc(xg, t1w[...], t1b)) * ibn * tg1[...] + tc1[...]
    z = jax.nn.relu(fc(z, t2w[...], t2b)) * ibn * tg2[...] + tc2[...]
    xs_ref[0] = lsm(fc(z, t3w[...], t3b))


def _tc(body, out_shape):
    return pl.pallas_call(body, out_shape=out_shape)


NBUF = 4


def _sc_scatter_body(hlin, srcr, dstr, zer, out, sidx, didx, rb0, rb1, rb2,
                     rb3, obuf, acc, gs0, gs1, gs2, gs3, ss0, ss1, ss2, ss3):
    c = lax.axis_index("c")
    s = lax.axis_index("s")
    w = s * NC + c
    row0 = s * RPT
    rbs = (rb0, rb1, rb2, rb3)
    gss = (gs0, gs1, gs2, gs3)
    sss = (ss0, ss1, ss2, ss3)
    pltpu.sync_copy(zer, obuf)
    pltpu.sync_copy(obuf, acc.at[pl.ds(row0, OBR)])
    pltpu.sync_copy(obuf, acc.at[pl.ds(row0 + OBR, OBR)])
    pltpu.sync_copy(srcr.at[pl.ds(w * WROWS, WROWS)], sidx)
    pltpu.sync_copy(dstr.at[pl.ds(w * WROWS, WROWS)], didx)
    plsc.subcore_barrier()

    for k in range(NBUF):
        pltpu.async_copy(hlin.at[sidx.at[k]], rbs[k], gss[k])

    def blk(t, carry):
        i = NBUF * t
        for k in range(NBUF):
            pltpu.make_async_copy(hlin.at[sidx.at[i + k]], rbs[k],
                                  gss[k]).wait()
            pltpu.async_copy(rbs[k], acc.at[didx.at[i + k]], sss[k], add=True)
        for k in range(NBUF):
            pltpu.make_async_copy(rbs[k], acc.at[didx.at[i + k]],
                                  sss[k]).wait()

            @pl.when(i + NBUF + k < WROWS)
            def _():
                pltpu.async_copy(hlin.at[sidx.at[i + NBUF + k]], rbs[k],
                                 gss[k])

        return carry

    lax.fori_loop(0, WROWS // NBUF, blk, 0)
    plsc.subcore_barrier()
    pltpu.sync_copy(acc.at[pl.ds(row0, OBR)], obuf)
    pltpu.sync_copy(obuf, out.at[c, pl.ds(row0, OBR)])
    pltpu.sync_copy(acc.at[pl.ds(row0 + OBR, OBR)], obuf)
    pltpu.sync_copy(obuf, out.at[c, pl.ds(row0 + OBR, OBR)])


def _scatter_edges(hlin, src2d, dst2d, zer):
    mesh = plsc.VectorSubcoreMesh(core_axis_name="c", subcore_axis_name="s")
    f = pl.kernel(
        _sc_scatter_body,
        out_type=jax.ShapeDtypeStruct((NC, ACCR, DIM), jnp.float32),
        mesh=mesh,
        scratch_types=[
            pltpu.VMEM((WROWS, ROWLEN), jnp.int32),
            pltpu.VMEM((WROWS, ROWLEN), jnp.int32),
            pltpu.VMEM((ROWLEN, DIM), jnp.float32),
            pltpu.VMEM((ROWLEN, DIM), jnp.float32),
            pltpu.VMEM((ROWLEN, DIM), jnp.float32),
            pltpu.VMEM((ROWLEN, DIM), jnp.float32),
            pltpu.VMEM((OBR, DIM), jnp.float32),
            pltpu.VMEM_SHARED((ACCR, DIM), jnp.float32),
            pltpu.SemaphoreType.DMA,
            pltpu.SemaphoreType.DMA,
            pltpu.SemaphoreType.DMA,
            pltpu.SemaphoreType.DMA,
            pltpu.SemaphoreType.DMA,
            pltpu.SemaphoreType.DMA,
            pltpu.SemaphoreType.DMA,
            pltpu.SemaphoreType.DMA,
        ],
        compiler_params=pltpu.CompilerParams(use_tc_tiling_on_sc=False),
    )
    return f(hlin, src2d, dst2d, zer)


def kernel(x, edge_index, edge_attr, batch, W1, b1, p1w, W2, b2, p2w,
           fc1W, fc1b, bn1g, bn1b, fc2W, fc2b, bn2g, bn2b, fc3W, fc3b,
           sW1, sb1, sg1, sbb1, sW2, sb2, sg2, sbb2, sW3, sb3):
    src2d = edge_index[0].reshape(EROWS, ROWLEN)
    dst2d = edge_index[1].reshape(EROWS, ROWLEN)
    zer = jnp.zeros((OBR, DIM), jnp.float32)
    r1 = lambda v: v.reshape(1, -1)

    hlin1 = _tc(_mm1_body, jax.ShapeDtypeStruct((N, DIM), jnp.float32))(x, W1)

    parts1 = _scatter_edges(hlin1, src2d, dst2d, zer)

    def _full(shape):
        return pl.BlockSpec(shape, lambda i: tuple(0 for _ in shape))

    def _gblk(minor):
        return pl.BlockSpec((1, GPB, minor), lambda i: (i, 0, 0))

    ap_spec = pl.BlockSpec((2, RPB, DIM), lambda i: (0, i, 0))

    vals1, selm1, sc1, x1, hlin2 = pl.pallas_call(
        _stage1_body,
        grid=(GB,),
        in_specs=[ap_spec, _full((1, DIM)), _full((1, DIM)),
                  _full((DIM, DIM))],
        out_specs=[_gblk(K1), _gblk(NPG), _gblk(NPG), _gblk(2 * DIM),
                   pl.BlockSpec((RPB, DIM), lambda i: (i, 0))],
        out_shape=(
            jax.ShapeDtypeStruct((GB, GPB, K1), jnp.float32),
            jax.ShapeDtypeStruct((GB, GPB, NPG), jnp.float32),
            jax.ShapeDtypeStruct((GB, GPB, NPG), jnp.float32),
            jax.ShapeDtypeStruct((GB, GPB, 2 * DIM), jnp.float32),
            jax.ShapeDtypeStruct((N, DIM), jnp.float32)))(
        parts1, r1(b1), r1(p1w), W2)

    parts2 = _scatter_edges(hlin2, src2d, dst2d, zer)

    vals2, xy, xs = pl.pallas_call(
        _stage2_body,
        grid=(GB,),
        scratch_shapes=[pltpu.VMEM((GPB, NPG), jnp.float32)],
        in_specs=[ap_spec, _full((1, DIM)), _full((1, DIM)),
                  _gblk(NPG), _gblk(NPG), _gblk(2 * DIM),
                  _full(fc1W.shape), _full((1, fc1b.shape[0])),
                  _full((1, bn1g.shape[0])), _full((1, bn1b.shape[0])),
                  _full(fc2W.shape), _full((1, fc2b.shape[0])),
                  _full((1, bn2g.shape[0])), _full((1, bn2b.shape[0])),
                  _full(fc3W.shape), _full((1, fc3b.shape[0])),
                  _full(sW1.shape), _full((1, sb1.shape[0])),
                  _full((1, sg1.shape[0])), _full((1, sbb1.shape[0])),
                  _full(sW2.shape), _full((1, sb2.shape[0])),
                  _full((1, sg2.shape[0])), _full((1, sbb2.shape[0])),
                  _full(sW3.shape), _full((1, sb3.shape[0]))],
        out_specs=[_gblk(K2), _gblk(2), _gblk(20)],
        out_shape=(
            jax.ShapeDtypeStruct((GB, GPB, K2), jnp.float32),
            jax.ShapeDtypeStruct((GB, GPB, 2), jnp.float32),
            jax.ShapeDtypeStruct((GB, GPB, 20), jnp.float32)))(
        parts2, r1(b2), r1(p2w), selm1, sc1, x1,
        fc1W, r1(fc1b), r1(bn1g), r1(bn1b),
        fc2W, r1(fc2b), r1(bn2g), r1(bn2b), fc3W, r1(fc3b),
        sW1, r1(sb1), r1(sg1), r1(sbb1),
        sW2, r1(sb2), r1(sg2), r1(sbb2), sW3, r1(sb3))

    return (xy.reshape(G, 2), xs.reshape(G, 20),
            vals1.reshape(G, K1), vals2.reshape(G, K2))
